# baseline (speedup 1.0000x reference)
import jax
import jax.numpy as jnp
from jax import lax
from jax.experimental import pallas as pl
from jax.experimental.pallas import tpu as pltpu
from jax.experimental.pallas import tpu_sc as plsc

BATCH = 4096
SEQ_LEN = 200
EMBED_DIM = 32
B_TOTAL = BATCH * SEQ_LEN
NUM_WORKERS = 32
CHUNK = 512
NCH_W = (B_TOTAL // CHUNK) // NUM_WORKERS


def _gather_pack_kernel(table_hbm, idx_hbm, out_hbm, idx_v, rows_v, pk_v,
                        sem_i0, sem_i1, sem_g0, sem_g1, sem_o0, sem_o1):
    sems_i = (sem_i0, sem_i1)
    sems_g = (sem_g0, sem_g1)
    sems_o = (sem_o0, sem_o1)

    wid = lax.axis_index("s") * 2 + lax.axis_index("c")
    c0 = wid * NCH_W
    clast = c0 + NCH_W - 1

    iota = lax.iota(jnp.int32, 16)

    def start_idx(c, b):
        pltpu.async_copy(idx_hbm.at[pl.ds(c * CHUNK, CHUNK)], idx_v.at[b],
                         sems_i[b])

    def wait_idx(c, b):
        pltpu.make_async_copy(idx_hbm.at[pl.ds(c * CHUNK, CHUNK)],
                              idx_v.at[b], sems_i[b]).wait()

    def start_gather(b):
        pltpu.async_copy(table_hbm.at[idx_v.at[b]], rows_v.at[b], sems_g[b])

    def wait_gather(b):
        pltpu.make_async_copy(table_hbm.at[idx_v.at[b]], rows_v.at[b],
                              sems_g[b]).wait()

    def pack(b):
        def d_step(d, carry):
            dg = d // 8
            dl = d - dg * 8
            col = jnp.full((16,), d, jnp.int32)
            base = dg * 4096 + dl * 128
            for k in range(4):
                for m in range(8):
                    row = iota + (k * 128 + m * 16)
                    x = plsc.load_gather(rows_v.at[b], [row, col])
                    pk_v[b, pl.ds(base + k * 1024 + m * 16, 16)] = x
            return carry

        lax.fori_loop(0, 32, d_step, 0)

    def start_wb(c, b):
        l = c // 8
        bg0 = (c % 8) * 4
        for dg in range(4):
            r0 = ((l * 4 + dg) * 32 + bg0) * 8
            pltpu.async_copy(pk_v.at[b, pl.ds(dg * 4096, 4096)],
                             out_hbm.at[pl.ds(r0 * 128, 4096)], sems_o[b])

    def wait_wb(c, b):
        l = c // 8
        bg0 = (c % 8) * 4
        for dg in range(4):
            r0 = ((l * 4 + dg) * 32 + bg0) * 8
            pltpu.make_async_copy(pk_v.at[b, pl.ds(dg * 4096, 4096)],
                                  out_hbm.at[pl.ds(r0 * 128, 4096)],
                                  sems_o[b]).wait()

    start_idx(c0, 0)
    start_idx(c0 + 1, 1)
    wait_idx(c0, 0)
    start_gather(0)
    wait_gather(0)
    wait_idx(c0 + 1, 1)
    start_gather(1)
    pack(0)
    start_idx(c0 + 2, 0)
    start_wb(c0, 0)

    def body(i, b):
        c = c0 + i
        wait_gather(b)
        wait_idx(c + 1, 1 - b)
        wait_wb(c - 1, 1 - b)
        start_gather(1 - b)
        pack(b)
        start_idx(jnp.minimum(c + 2, clast), b)
        start_wb(c, b)

    def pair(p, carry):
        body(2 * p + 1, 1)
        body(2 * p + 2, 0)
        return carry

    lax.fori_loop(0, (NCH_W - 2) // 2, pair, 0)

    wait_gather(1)
    pack(1)
    start_wb(clast, 1)

    wait_wb(clast - 1, 0)
    wait_wb(clast, 1)
    wait_idx(clast, 0)


@jax.jit
def kernel(seq, emb_table):
    flat_idx = seq.T.reshape(B_TOTAL)
    call = pl.kernel(
        _gather_pack_kernel,
        out_type=jax.ShapeDtypeStruct((B_TOTAL * EMBED_DIM,), jnp.float32),
        mesh=plsc.VectorSubcoreMesh(core_axis_name="c", subcore_axis_name="s"),
        scratch_types=[
            pltpu.VMEM((2, CHUNK), jnp.int32),
            pltpu.VMEM((2, CHUNK, EMBED_DIM), jnp.float32),
            pltpu.VMEM((2, CHUNK * EMBED_DIM), jnp.float32),
        ] + [pltpu.SemaphoreType.DMA] * 6,
        compiler_params=pltpu.CompilerParams(
            use_tc_tiling_on_sc=False, needs_layout_passes=False),
    )
    out = call(emb_table, flat_idx)
    out5 = out.reshape(SEQ_LEN, 4, 32, 8, 128)
    return out5.transpose(2, 4, 0, 1, 3).reshape(BATCH, SEQ_LEN, EMBED_DIM)


# speedup vs baseline: 1.1283x; 1.1283x over previous
import jax
import jax.numpy as jnp
from jax import lax
from jax.experimental import pallas as pl
from jax.experimental.pallas import tpu as pltpu
from jax.experimental.pallas import tpu_sc as plsc

BATCH = 4096
SEQ_LEN = 200
EMBED_DIM = 32
B_TOTAL = BATCH * SEQ_LEN
NUM_WORKERS = 32
CHUNK = 512
NCH_W = (B_TOTAL // CHUNK) // NUM_WORKERS


def _gather_pack_kernel(table_hbm, idx_hbm, out_hbm, idx_v, rows_v, pk_v,
                        sem_i0, sem_i1, sem_g0, sem_g1, sem_o0, sem_o1):
    sems_i = (sem_i0, sem_i1)
    sems_g = (sem_g0, sem_g1)
    sems_o = (sem_o0, sem_o1)

    wid = lax.axis_index("s") * 2 + lax.axis_index("c")
    c0 = wid * NCH_W
    clast = c0 + NCH_W - 1

    iota = lax.iota(jnp.int32, 16)

    def start_idx(c, b):
        pltpu.async_copy(idx_hbm.at[pl.ds(c * CHUNK, CHUNK)], idx_v.at[b],
                         sems_i[b])

    def wait_idx(c, b):
        pltpu.make_async_copy(idx_hbm.at[pl.ds(c * CHUNK, CHUNK)],
                              idx_v.at[b], sems_i[b]).wait()

    def start_gather(b):
        pltpu.async_copy(table_hbm.at[idx_v.at[b]], rows_v.at[b], sems_g[b])

    def wait_gather(b):
        pltpu.make_async_copy(table_hbm.at[idx_v.at[b]], rows_v.at[b],
                              sems_g[b]).wait()

    base_lo = (iota // 8) * 4096 + (iota % 8) * 128
    base_hi = base_lo + 8192

    def pack(b):
        def row_step(j0, carry):
            for u in range(8):
                j = j0 * 8 + u
                c = j + (j // 128) * 896
                x0 = rows_v[b, j, pl.ds(0, 16)]
                x1 = rows_v[b, j, pl.ds(16, 16)]
                plsc.store_scatter(pk_v.at[b], [base_lo + c], x0)
                plsc.store_scatter(pk_v.at[b], [base_hi + c], x1)
            return carry

        lax.fori_loop(0, CHUNK // 8, row_step, 0)

    def start_wb(c, b):
        l = c // 8
        bg0 = (c % 8) * 4
        for dg in range(4):
            r0 = ((l * 4 + dg) * 32 + bg0) * 8
            pltpu.async_copy(pk_v.at[b, pl.ds(dg * 4096, 4096)],
                             out_hbm.at[pl.ds(r0 * 128, 4096)], sems_o[b])

    def wait_wb(c, b):
        l = c // 8
        bg0 = (c % 8) * 4
        for dg in range(4):
            r0 = ((l * 4 + dg) * 32 + bg0) * 8
            pltpu.make_async_copy(pk_v.at[b, pl.ds(dg * 4096, 4096)],
                                  out_hbm.at[pl.ds(r0 * 128, 4096)],
                                  sems_o[b]).wait()

    start_idx(c0, 0)
    start_idx(c0 + 1, 1)
    wait_idx(c0, 0)
    start_gather(0)
    wait_gather(0)
    wait_idx(c0 + 1, 1)
    start_gather(1)
    pack(0)
    start_idx(c0 + 2, 0)
    start_wb(c0, 0)

    def body(i, b):
        c = c0 + i
        wait_gather(b)
        wait_idx(c + 1, 1 - b)
        wait_wb(c - 1, 1 - b)
        start_gather(1 - b)
        pack(b)
        start_idx(jnp.minimum(c + 2, clast), b)
        start_wb(c, b)

    def pair(p, carry):
        body(2 * p + 1, 1)
        body(2 * p + 2, 0)
        return carry

    lax.fori_loop(0, (NCH_W - 2) // 2, pair, 0)

    wait_gather(1)
    pack(1)
    start_wb(clast, 1)

    wait_wb(clast - 1, 0)
    wait_wb(clast, 1)
    wait_idx(clast, 0)


@jax.jit
def kernel(seq, emb_table):
    flat_idx = seq.T.reshape(B_TOTAL)
    call = pl.kernel(
        _gather_pack_kernel,
        out_type=jax.ShapeDtypeStruct((B_TOTAL * EMBED_DIM,), jnp.float32),
        mesh=plsc.VectorSubcoreMesh(core_axis_name="c", subcore_axis_name="s"),
        scratch_types=[
            pltpu.VMEM((2, CHUNK), jnp.int32),
            pltpu.VMEM((2, CHUNK, EMBED_DIM), jnp.float32),
            pltpu.VMEM((2, CHUNK * EMBED_DIM), jnp.float32),
        ] + [pltpu.SemaphoreType.DMA] * 6,
        compiler_params=pltpu.CompilerParams(
            use_tc_tiling_on_sc=False, needs_layout_passes=False),
    )
    out = call(emb_table, flat_idx)
    out5 = out.reshape(SEQ_LEN, 4, 32, 8, 128)
    return out5.transpose(2, 4, 0, 1, 3).reshape(BATCH, SEQ_LEN, EMBED_DIM)


# speedup vs baseline: 1.2709x; 1.1263x over previous
import jax
import jax.numpy as jnp
from jax import lax
from jax.experimental import pallas as pl
from jax.experimental.pallas import tpu as pltpu
from jax.experimental.pallas import tpu_sc as plsc

BATCH = 4096
SEQ_LEN = 200
EMBED_DIM = 32
B_TOTAL = BATCH * SEQ_LEN
NUM_WORKERS = 32
CHUNK = 512
NCH_W = (B_TOTAL // CHUNK) // NUM_WORKERS


def _gather_pack_kernel(table_hbm, idx_hbm, out_hbm, idx_v, rows_v, pk_v,
                        sem_i0, sem_i1, sem_g0, sem_g1, sem_o0, sem_o1):
    sems_i = (sem_i0, sem_i1)
    sems_g = (sem_g0, sem_g1)
    sems_o = (sem_o0, sem_o1)

    wid = lax.axis_index("s") * 2 + lax.axis_index("c")
    c0 = wid * NCH_W
    clast = c0 + NCH_W - 1

    iota = lax.iota(jnp.int32, 16)

    def start_idx(c, b):
        pltpu.async_copy(idx_hbm.at[pl.ds(c * CHUNK, CHUNK)], idx_v.at[b],
                         sems_i[b])

    def wait_idx(c, b):
        pltpu.make_async_copy(idx_hbm.at[pl.ds(c * CHUNK, CHUNK)],
                              idx_v.at[b], sems_i[b]).wait()

    def start_gather(b):
        pltpu.async_copy(table_hbm.at[idx_v.at[b]], rows_v.at[b], sems_g[b])

    def wait_gather(b):
        pltpu.make_async_copy(table_hbm.at[idx_v.at[b]], rows_v.at[b],
                              sems_g[b]).wait()

    base_lo = (iota // 8) * 4096 + (iota % 8) * 128
    base_hi = base_lo + 8192

    def pack(b):
        @plsc.parallel_loop(0, CHUNK, unroll=8)
        def _row(j):
            c = j + (j // 128) * 896
            x0 = rows_v[b, j, pl.ds(0, 16)]
            x1 = rows_v[b, j, pl.ds(16, 16)]
            plsc.store_scatter(pk_v.at[b], [base_lo + c], x0)
            plsc.store_scatter(pk_v.at[b], [base_hi + c], x1)

    def start_wb(c, b):
        l = c // 8
        bg0 = (c % 8) * 4
        for dg in range(4):
            r0 = ((l * 4 + dg) * 32 + bg0) * 8
            pltpu.async_copy(pk_v.at[b, pl.ds(dg * 4096, 4096)],
                             out_hbm.at[pl.ds(r0 * 128, 4096)], sems_o[b])

    def wait_wb(c, b):
        l = c // 8
        bg0 = (c % 8) * 4
        for dg in range(4):
            r0 = ((l * 4 + dg) * 32 + bg0) * 8
            pltpu.make_async_copy(pk_v.at[b, pl.ds(dg * 4096, 4096)],
                                  out_hbm.at[pl.ds(r0 * 128, 4096)],
                                  sems_o[b]).wait()

    start_idx(c0, 0)
    start_idx(c0 + 1, 1)
    wait_idx(c0, 0)
    start_gather(0)
    wait_gather(0)
    wait_idx(c0 + 1, 1)
    start_gather(1)
    pack(0)
    start_idx(c0 + 2, 0)
    start_wb(c0, 0)

    def body(i, b):
        c = c0 + i
        wait_gather(b)
        wait_idx(c + 1, 1 - b)
        wait_wb(c - 1, 1 - b)
        start_gather(1 - b)
        pack(b)
        start_idx(jnp.minimum(c + 2, clast), b)
        start_wb(c, b)

    def pair(p, carry):
        body(2 * p + 1, 1)
        body(2 * p + 2, 0)
        return carry

    lax.fori_loop(0, (NCH_W - 2) // 2, pair, 0)

    wait_gather(1)
    pack(1)
    start_wb(clast, 1)

    wait_wb(clast - 1, 0)
    wait_wb(clast, 1)
    wait_idx(clast, 0)


@jax.jit
def kernel(seq, emb_table):
    flat_idx = seq.T.reshape(B_TOTAL)
    call = pl.kernel(
        _gather_pack_kernel,
        out_type=jax.ShapeDtypeStruct((B_TOTAL * EMBED_DIM,), jnp.float32),
        mesh=plsc.VectorSubcoreMesh(core_axis_name="c", subcore_axis_name="s"),
        scratch_types=[
            pltpu.VMEM((2, CHUNK), jnp.int32),
            pltpu.VMEM((2, CHUNK, EMBED_DIM), jnp.float32),
            pltpu.VMEM((2, CHUNK * EMBED_DIM), jnp.float32),
        ] + [pltpu.SemaphoreType.DMA] * 6,
        compiler_params=pltpu.CompilerParams(
            use_tc_tiling_on_sc=False, needs_layout_passes=False),
    )
    out = call(emb_table, flat_idx)
    out5 = out.reshape(SEQ_LEN, 4, 32, 8, 128)
    return out5.transpose(2, 4, 0, 1, 3).reshape(BATCH, SEQ_LEN, EMBED_DIM)


# speedup vs baseline: 1.9495x; 1.5340x over previous
import jax
import jax.numpy as jnp
from jax import lax
from jax.experimental import pallas as pl
from jax.experimental.pallas import tpu as pltpu
from jax.experimental.pallas import tpu_sc as plsc

BATCH = 4096
SEQ_LEN = 200
EMBED_DIM = 32
B_TOTAL = BATCH * SEQ_LEN
NUM_WORKERS = 32
CHUNK = 512
NCH_W = (B_TOTAL // CHUNK) // NUM_WORKERS


def _gather_pack_kernel(table_hbm, idx_hbm, out_hbm, idx_v, rows_v, pk_v,
                        sem_i0, sem_i1, sem_g0, sem_g1, sem_o0, sem_o1):
    sems_i = (sem_i0, sem_i1)
    sems_g = (sem_g0, sem_g1)
    sems_o = (sem_o0, sem_o1)

    wid = lax.axis_index("s") * 2 + lax.axis_index("c")
    c0 = wid * NCH_W
    clast = c0 + NCH_W - 1

    iota = lax.iota(jnp.int32, 16)

    def start_idx(c, b):
        pltpu.async_copy(idx_hbm.at[pl.ds(c * CHUNK, CHUNK)], idx_v.at[b],
                         sems_i[b])

    def wait_idx(c, b):
        pltpu.make_async_copy(idx_hbm.at[pl.ds(c * CHUNK, CHUNK)],
                              idx_v.at[b], sems_i[b]).wait()

    def start_gather(b):
        pltpu.async_copy(table_hbm.at[idx_v.at[b]], rows_v.at[b], sems_g[b])

    def wait_gather(b):
        pltpu.make_async_copy(table_hbm.at[idx_v.at[b]], rows_v.at[b],
                              sems_g[b]).wait()

    rv_lo = (iota // 8) * 8 + (iota % 8)
    rv_hi = rv_lo + 16

    def pack(b):
        for k in range(4):
            rlo = rv_lo + k * 32
            rhi = rv_hi + k * 32

            @plsc.parallel_loop(0, 128, unroll=8)
            def _col(bl):
                col = jnp.full((16,), 0, jnp.int32) + bl
                x0 = rows_v[b, k * 128 + bl, pl.ds(0, 16)]
                x1 = rows_v[b, k * 128 + bl, pl.ds(16, 16)]
                plsc.store_scatter(pk_v.at[b], [rlo, col], x0)
                plsc.store_scatter(pk_v.at[b], [rhi, col], x1)

    def _wb_pairs(c, b):
        l = c // 8
        bg0 = (c % 8) * 4
        for dg in range(4):
            r0 = ((l * 4 + dg) * 32 + bg0) * 8
            for k in range(4):
                src = pk_v.at[b, pl.ds(k * 32 + dg * 8, 8), pl.ds(0, 128)]
                dst = out_hbm.at[pl.ds(r0 + k * 8, 8), pl.ds(0, 128)]
                yield src, dst

    def start_wb(c, b):
        for src, dst in _wb_pairs(c, b):
            pltpu.async_copy(src, dst, sems_o[b])

    def wait_wb(c, b):
        for src, dst in _wb_pairs(c, b):
            pltpu.make_async_copy(src, dst, sems_o[b]).wait()

    start_idx(c0, 0)
    start_idx(c0 + 1, 1)
    wait_idx(c0, 0)
    start_gather(0)
    wait_gather(0)
    wait_idx(c0 + 1, 1)
    start_gather(1)
    pack(0)
    start_idx(c0 + 2, 0)
    start_wb(c0, 0)

    def body(i, b):
        c = c0 + i
        wait_gather(b)
        wait_idx(c + 1, 1 - b)
        wait_wb(c - 1, 1 - b)
        start_gather(1 - b)
        pack(b)
        start_idx(jnp.minimum(c + 2, clast), b)
        start_wb(c, b)

    def pair(p, carry):
        body(2 * p + 1, 1)
        body(2 * p + 2, 0)
        return carry

    lax.fori_loop(0, (NCH_W - 2) // 2, pair, 0)

    wait_gather(1)
    pack(1)
    start_wb(clast, 1)

    wait_wb(clast - 1, 0)
    wait_wb(clast, 1)
    wait_idx(clast, 0)


@jax.jit
def kernel(seq, emb_table):
    flat_idx = seq.T.reshape(B_TOTAL)
    call = pl.kernel(
        _gather_pack_kernel,
        out_type=jax.ShapeDtypeStruct((B_TOTAL * EMBED_DIM // 128, 128),
                                      jnp.float32),
        mesh=plsc.VectorSubcoreMesh(core_axis_name="c", subcore_axis_name="s"),
        scratch_types=[
            pltpu.VMEM((2, CHUNK), jnp.int32),
            pltpu.VMEM((2, CHUNK, EMBED_DIM), jnp.float32),
            pltpu.VMEM((2, 128, 129), jnp.float32),
        ] + [pltpu.SemaphoreType.DMA] * 6,
        compiler_params=pltpu.CompilerParams(
            use_tc_tiling_on_sc=False, needs_layout_passes=False),
    )
    out = call(emb_table, flat_idx)
    out5 = out.reshape(SEQ_LEN, 4, 32, 8, 128)
    return out5.transpose(2, 4, 0, 1, 3).reshape(BATCH, SEQ_LEN, EMBED_DIM)
